# baseline (speedup 1.0000x reference)
import functools

import jax
import jax.numpy as jnp
from jax import lax
from jax.experimental import pallas as pl
from jax.experimental.pallas import tpu as pltpu
from jax.experimental.pallas import tpu_sc as plsc

N = 10000
E = 320000
D = 128

NC = 2
NS = 16
NW = NC * NS

EPW = E // NW
CH = 128
NCH = 80
EPWP = NCH * CH
PAD = EPWP - EPW
NROWS = N + PAD
RPT = 624
REM = N - NS * RPT
REM_OFF = NS * RPT


def _sc_aggregate(h, srcp, dstp):
  mesh = plsc.VectorSubcoreMesh(core_axis_name="c", subcore_axis_name="s")

  @functools.partial(
      pl.kernel,
      mesh=mesh,
      out_type=jax.ShapeDtypeStruct((NC, N, D), jnp.float32),
      scratch_types=[
          pltpu.VMEM((EPWP,), jnp.int32),
          pltpu.VMEM((CH,), jnp.int32),
          pltpu.VMEM((CH,), jnp.int32),
          pltpu.VMEM((CH, D), jnp.float32),
          pltpu.VMEM((CH, D), jnp.float32),
          pltpu.VMEM_SHARED((NROWS, D), jnp.float32),
          pltpu.SemaphoreType.DMA,
          pltpu.SemaphoreType.DMA,
          pltpu.SemaphoreType.DMA,
          pltpu.SemaphoreType.DMA,
      ],
  )
  def agg_kernel(h_hbm, src_hbm, dst_hbm, out_hbm, srcall_v,
                 dsta_v, dstb_v, rows_a, rows_b, acc_sh,
                 sem_a, sem_b, sem_da, sem_db):
    cid = lax.axis_index("c")
    sid = lax.axis_index("s")
    wid = sid * NC + cid

    pltpu.sync_copy(h_hbm.at[pl.ds(sid * RPT, RPT)],
                    acc_sh.at[pl.ds(sid * RPT, RPT)])

    @pl.when(sid == 0)
    def _():
      pltpu.sync_copy(h_hbm.at[pl.ds(REM_OFF, REM)],
                      acc_sh.at[pl.ds(REM_OFF, REM)])

    ebase = wid * EPWP
    pltpu.sync_copy(src_hbm.at[pl.ds(ebase, EPWP)], srcall_v)
    plsc.subcore_barrier()

    def gather(c, rows, sem):
      pltpu.async_copy(h_hbm.at[srcall_v.at[pl.ds(c * CH, CH)]], rows, sem)

    def dfetch(c, dstv, sem):
      pltpu.async_copy(dst_hbm.at[pl.ds(ebase + c * CH, CH)], dstv, sem)

    def scat(rows, dstv, sem, dsem):
      pltpu.make_async_copy(dst_hbm.at[pl.ds(0, CH)], dstv, dsem).wait()
      pltpu.make_async_copy(h_hbm.at[pl.ds(0, CH)], rows, sem).wait()
      pltpu.sync_copy(rows, acc_sh.at[dstv], add=True)

    dfetch(0, dsta_v, sem_da)
    gather(0, rows_a, sem_a)
    dfetch(1, dstb_v, sem_db)
    gather(1, rows_b, sem_b)

    def body(g, carry):
      c = 2 * g
      scat(rows_a, dsta_v, sem_a, sem_da)
      dfetch(c + 2, dsta_v, sem_da)
      gather(c + 2, rows_a, sem_a)
      scat(rows_b, dstb_v, sem_b, sem_db)
      dfetch(c + 3, dstb_v, sem_db)
      gather(c + 3, rows_b, sem_b)
      return carry

    lax.fori_loop(0, NCH // 2 - 1, body, 0)
    scat(rows_a, dsta_v, sem_a, sem_da)
    scat(rows_b, dstb_v, sem_b, sem_db)
    plsc.subcore_barrier()

    pltpu.sync_copy(acc_sh.at[pl.ds(sid * RPT, RPT)],
                    out_hbm.at[cid, pl.ds(sid * RPT, RPT)])

    @pl.when(sid == 0)
    def _():
      pltpu.sync_copy(acc_sh.at[pl.ds(REM_OFF, REM)],
                      out_hbm.at[cid, pl.ds(REM_OFF, REM)])

  return agg_kernel(h, srcp, dstp)


def _tc_layer(x, p, W, b, final):
  BR = 1000

  def body(x_ref, p_ref, w_ref, bias_ref, o_ref):
    rst = p_ref[0] + p_ref[1] - x_ref[...]
    y = jnp.dot(rst, w_ref[...], preferred_element_type=jnp.float32)
    y = jnp.maximum(y + bias_ref[...], 0.0)
    if final:
      m = jnp.max(y, axis=-1, keepdims=True)
      s = jnp.sum(jnp.exp(y - m), axis=-1, keepdims=True)
      y = y - (m + jnp.log(s))
    o_ref[...] = y

  row_spec = pl.BlockSpec((BR, D), lambda i: (i, 0))
  return pl.pallas_call(
      body,
      grid=(N // BR,),
      in_specs=[
          row_spec,
          pl.BlockSpec((NC, BR, D), lambda i: (0, i, 0)),
          pl.BlockSpec((D, D), lambda i: (0, 0)),
          pl.BlockSpec((1, D), lambda i: (0, 0)),
      ],
      out_specs=row_spec,
      out_shape=jax.ShapeDtypeStruct((N, D), jnp.float32),
  )(x, p, W, b)


def kernel(h, edge_index, W1, b1, W2, b2):
  src2 = edge_index[0].reshape(NW, EPW)
  dst2 = edge_index[1].reshape(NW, EPW)
  srcp = jnp.pad(src2, ((0, 0), (0, PAD))).reshape(-1)
  dump = jnp.broadcast_to(N + jnp.arange(PAD, dtype=jnp.int32), (NW, PAD))
  dstp = jnp.concatenate([dst2, dump], axis=1).reshape(-1)
  b1r = b1.reshape(1, D)
  b2r = b2.reshape(1, D)

  p = _sc_aggregate(h, srcp, dstp)
  h1 = _tc_layer(h, p, W1, b1r, final=False)
  p2 = _sc_aggregate(h1, srcp, dstp)
  return _tc_layer(h1, p2, W2, b2r, final=True)


# speedup vs baseline: 2.8985x; 2.8985x over previous
import functools

import jax
import jax.numpy as jnp
from jax import lax
from jax.experimental import pallas as pl
from jax.experimental.pallas import tpu as pltpu
from jax.experimental.pallas import tpu_sc as plsc

N = 10000
E = 320000
D = 128

NC = 2
NS = 16
NW = NC * NS

EPW = E // NW
CH = 80
NCH = 125
EPWP = NCH * CH
PAD = EPWP - EPW
NROWS = N
RPT = 624
REM = N - NS * RPT
REM_OFF = NS * RPT


def _sc_aggregate(h, srcp, dstp):
  mesh = plsc.VectorSubcoreMesh(core_axis_name="c", subcore_axis_name="s")

  @functools.partial(
      pl.kernel,
      mesh=mesh,
      out_type=jax.ShapeDtypeStruct((NC, N, D), jnp.float32),
      scratch_types=[
          pltpu.VMEM((EPWP,), jnp.int32),
          pltpu.VMEM((CH,), jnp.int32),
          pltpu.VMEM((CH,), jnp.int32),
          pltpu.VMEM((CH, D), jnp.float32),
          pltpu.VMEM((CH, D), jnp.float32),
          pltpu.VMEM_SHARED((NROWS, D), jnp.float32),
          pltpu.SemaphoreType.DMA,
          pltpu.SemaphoreType.DMA,
          pltpu.SemaphoreType.DMA,
          pltpu.SemaphoreType.DMA,
      ],
  )
  def agg_kernel(h_hbm, src_hbm, dst_hbm, out_hbm, srcall_v,
                 dsta_v, dstb_v, rows_a, rows_b, acc_sh,
                 sem_a, sem_b, sem_da, sem_db):
    cid = lax.axis_index("c")
    sid = lax.axis_index("s")
    wid = sid * NC + cid

    pltpu.sync_copy(h_hbm.at[pl.ds(sid * RPT, RPT)],
                    acc_sh.at[pl.ds(sid * RPT, RPT)])

    @pl.when(sid == 0)
    def _():
      pltpu.sync_copy(h_hbm.at[pl.ds(REM_OFF, REM)],
                      acc_sh.at[pl.ds(REM_OFF, REM)])

    ebase = wid * EPWP
    pltpu.sync_copy(src_hbm.at[pl.ds(ebase, EPWP)], srcall_v)
    plsc.subcore_barrier()

    def gather(c, rows, sem):
      pltpu.async_copy(h_hbm.at[srcall_v.at[pl.ds(c * CH, CH)]], rows, sem)

    def dfetch(c, dstv, sem):
      pltpu.async_copy(dst_hbm.at[pl.ds(ebase + c * CH, CH)], dstv, sem)

    def scat(rows, dstv, sem, dsem):
      pltpu.make_async_copy(dst_hbm.at[pl.ds(0, CH)], dstv, dsem).wait()
      pltpu.make_async_copy(h_hbm.at[pl.ds(0, CH)], rows, sem).wait()
      pltpu.sync_copy(rows, acc_sh.at[dstv], add=True)

    dfetch(0, dsta_v, sem_da)
    gather(0, rows_a, sem_a)

    def body(g, carry):
      c = 2 * g
      dfetch(c + 1, dstb_v, sem_db)
      gather(c + 1, rows_b, sem_b)
      scat(rows_a, dsta_v, sem_a, sem_da)
      dfetch(c + 2, dsta_v, sem_da)
      gather(c + 2, rows_a, sem_a)
      scat(rows_b, dstb_v, sem_b, sem_db)
      return carry

    lax.fori_loop(0, (NCH - 1) // 2, body, 0)
    scat(rows_a, dsta_v, sem_a, sem_da)
    plsc.subcore_barrier()

    pltpu.sync_copy(acc_sh.at[pl.ds(sid * RPT, RPT)],
                    out_hbm.at[cid, pl.ds(sid * RPT, RPT)])

    @pl.when(sid == 0)
    def _():
      pltpu.sync_copy(acc_sh.at[pl.ds(REM_OFF, REM)],
                      out_hbm.at[cid, pl.ds(REM_OFF, REM)])

  return agg_kernel(h, srcp, dstp)


def _tc_layer(x, p, W, b, final):
  BR = 1000

  def body(x_ref, p_ref, w_ref, bias_ref, o_ref):
    rst = p_ref[0] + p_ref[1] - x_ref[...]
    y = jnp.dot(rst, w_ref[...], preferred_element_type=jnp.float32)
    y = jnp.maximum(y + bias_ref[...], 0.0)
    if final:
      m = jnp.max(y, axis=-1, keepdims=True)
      s = jnp.sum(jnp.exp(y - m), axis=-1, keepdims=True)
      y = y - (m + jnp.log(s))
    o_ref[...] = y

  row_spec = pl.BlockSpec((BR, D), lambda i: (i, 0))
  return pl.pallas_call(
      body,
      grid=(N // BR,),
      in_specs=[
          row_spec,
          pl.BlockSpec((NC, BR, D), lambda i: (0, i, 0)),
          pl.BlockSpec((D, D), lambda i: (0, 0)),
          pl.BlockSpec((1, D), lambda i: (0, 0)),
      ],
      out_specs=row_spec,
      out_shape=jax.ShapeDtypeStruct((N, D), jnp.float32),
  )(x, p, W, b)


def kernel(h, edge_index, W1, b1, W2, b2):
  srcp = edge_index[0]
  dstp = edge_index[1]
  b1r = b1.reshape(1, D)
  b2r = b2.reshape(1, D)

  p = _sc_aggregate(h, srcp, dstp)
  h1 = _tc_layer(h, p, W1, b1r, final=False)
  p2 = _sc_aggregate(h1, srcp, dstp)
  return _tc_layer(h1, p2, W2, b2r, final=True)


# speedup vs baseline: 3.4183x; 1.1793x over previous
import functools

import jax
import jax.numpy as jnp
from jax import lax
from jax.experimental import pallas as pl
from jax.experimental.pallas import tpu as pltpu
from jax.experimental.pallas import tpu_sc as plsc

N = 10000
E = 320000
D = 128

NC = 2
NS = 16
NW = NC * NS

EPW = E // NW
CH = 80
NCH = 125
EPWP = NCH * CH
PAD = EPWP - EPW
NROWS = N
RPT = 624
REM = N - NS * RPT
REM_OFF = NS * RPT


def _sc_aggregate(h, srcp, dstp):
  mesh = plsc.VectorSubcoreMesh(core_axis_name="c", subcore_axis_name="s")

  @functools.partial(
      pl.kernel,
      mesh=mesh,
      out_type=jax.ShapeDtypeStruct((NC, N, D), jnp.float32),
      scratch_types=[
          pltpu.VMEM((EPWP,), jnp.int32),
          pltpu.VMEM((CH,), jnp.int32),
          pltpu.VMEM((CH,), jnp.int32),
          pltpu.VMEM((CH,), jnp.int32),
          pltpu.VMEM((CH, D), jnp.float32),
          pltpu.VMEM((CH, D), jnp.float32),
          pltpu.VMEM((CH, D), jnp.float32),
          pltpu.VMEM_SHARED((NROWS, D), jnp.float32),
          pltpu.SemaphoreType.DMA,
          pltpu.SemaphoreType.DMA,
          pltpu.SemaphoreType.DMA,
          pltpu.SemaphoreType.DMA,
          pltpu.SemaphoreType.DMA,
          pltpu.SemaphoreType.DMA,
      ],
  )
  def agg_kernel(h_hbm, src_hbm, dst_hbm, out_hbm, srcall_v,
                 dsta_v, dstb_v, dstc_v, rows_a, rows_b, rows_c, acc_sh,
                 sem_a, sem_b, sem_c, sem_da, sem_db, sem_dc):
    cid = lax.axis_index("c")
    sid = lax.axis_index("s")
    wid = sid * NC + cid

    pltpu.sync_copy(h_hbm.at[pl.ds(sid * RPT, RPT)],
                    acc_sh.at[pl.ds(sid * RPT, RPT)])

    @pl.when(sid == 0)
    def _():
      pltpu.sync_copy(h_hbm.at[pl.ds(REM_OFF, REM)],
                      acc_sh.at[pl.ds(REM_OFF, REM)])

    ebase = wid * EPWP
    pltpu.sync_copy(src_hbm.at[pl.ds(ebase, EPWP)], srcall_v)
    plsc.subcore_barrier()

    def gather(c, rows, sem):
      pltpu.async_copy(h_hbm.at[srcall_v.at[pl.ds(c * CH, CH)]], rows, sem)

    def dfetch(c, dstv, sem):
      pltpu.async_copy(dst_hbm.at[pl.ds(ebase + c * CH, CH)], dstv, sem)

    def scat(rows, dstv, sem, dsem):
      pltpu.make_async_copy(dst_hbm.at[pl.ds(0, CH)], dstv, dsem).wait()
      pltpu.make_async_copy(h_hbm.at[pl.ds(0, CH)], rows, sem).wait()
      pltpu.sync_copy(rows, acc_sh.at[dstv], add=True)

    dfetch(0, dsta_v, sem_da)
    gather(0, rows_a, sem_a)
    dfetch(1, dstb_v, sem_db)
    gather(1, rows_b, sem_b)

    def body(g, carry):
      c = 3 * g
      dfetch(c + 2, dstc_v, sem_dc)
      gather(c + 2, rows_c, sem_c)
      scat(rows_a, dsta_v, sem_a, sem_da)
      dfetch(c + 3, dsta_v, sem_da)
      gather(c + 3, rows_a, sem_a)
      scat(rows_b, dstb_v, sem_b, sem_db)
      dfetch(c + 4, dstb_v, sem_db)
      gather(c + 4, rows_b, sem_b)
      scat(rows_c, dstc_v, sem_c, sem_dc)
      return carry

    lax.fori_loop(0, NCH // 3, body, 0)
    scat(rows_a, dsta_v, sem_a, sem_da)
    scat(rows_b, dstb_v, sem_b, sem_db)
    plsc.subcore_barrier()

    pltpu.sync_copy(acc_sh.at[pl.ds(sid * RPT, RPT)],
                    out_hbm.at[cid, pl.ds(sid * RPT, RPT)])

    @pl.when(sid == 0)
    def _():
      pltpu.sync_copy(acc_sh.at[pl.ds(REM_OFF, REM)],
                      out_hbm.at[cid, pl.ds(REM_OFF, REM)])

  return agg_kernel(h, srcp, dstp)


def _tc_layer(x, p, W, b, final):
  BR = 1000

  def body(x_ref, p_ref, w_ref, bias_ref, o_ref):
    rst = p_ref[0] + p_ref[1] - x_ref[...]
    y = jnp.dot(rst, w_ref[...], preferred_element_type=jnp.float32)
    y = jnp.maximum(y + bias_ref[...], 0.0)
    if final:
      m = jnp.max(y, axis=-1, keepdims=True)
      s = jnp.sum(jnp.exp(y - m), axis=-1, keepdims=True)
      y = y - (m + jnp.log(s))
    o_ref[...] = y

  row_spec = pl.BlockSpec((BR, D), lambda i: (i, 0))
  return pl.pallas_call(
      body,
      grid=(N // BR,),
      in_specs=[
          row_spec,
          pl.BlockSpec((NC, BR, D), lambda i: (0, i, 0)),
          pl.BlockSpec((D, D), lambda i: (0, 0)),
          pl.BlockSpec((1, D), lambda i: (0, 0)),
      ],
      out_specs=row_spec,
      out_shape=jax.ShapeDtypeStruct((N, D), jnp.float32),
  )(x, p, W, b)


def kernel(h, edge_index, W1, b1, W2, b2):
  srcp = edge_index[0]
  dstp = edge_index[1]
  b1r = b1.reshape(1, D)
  b2r = b2.reshape(1, D)

  p = _sc_aggregate(h, srcp, dstp)
  h1 = _tc_layer(h, p, W1, b1r, final=False)
  p2 = _sc_aggregate(h1, srcp, dstp)
  return _tc_layer(h1, p2, W2, b2r, final=True)


# speedup vs baseline: 3.5464x; 1.0375x over previous
import functools

import jax
import jax.numpy as jnp
from jax import lax
from jax.experimental import pallas as pl
from jax.experimental.pallas import tpu as pltpu
from jax.experimental.pallas import tpu_sc as plsc

N = 10000
E = 320000
D = 128

NC = 2
NS = 16
NW = NC * NS

EPW = E // NW
CH = 80
NCH = 125
EPWP = NCH * CH
PAD = EPWP - EPW
NROWS = N
RPT = 624
REM = N - NS * RPT
REM_OFF = NS * RPT


def _sc_aggregate(h, srcp, dstp):
  mesh = plsc.VectorSubcoreMesh(core_axis_name="c", subcore_axis_name="s")

  @functools.partial(
      pl.kernel,
      mesh=mesh,
      out_type=jax.ShapeDtypeStruct((NC, N, D), jnp.float32),
      scratch_types=[
          pltpu.VMEM((EPWP,), jnp.int32),
          pltpu.VMEM((CH,), jnp.int32),
          pltpu.VMEM((CH,), jnp.int32),
          pltpu.VMEM((CH,), jnp.int32),
          pltpu.VMEM((CH, D), jnp.float32),
          pltpu.VMEM((CH, D), jnp.float32),
          pltpu.VMEM((CH, D), jnp.float32),
          pltpu.VMEM_SHARED((NROWS, D), jnp.float32),
          pltpu.SemaphoreType.DMA,
          pltpu.SemaphoreType.DMA,
          pltpu.SemaphoreType.DMA,
          pltpu.SemaphoreType.DMA,
          pltpu.SemaphoreType.DMA,
          pltpu.SemaphoreType.DMA,
      ],
  )
  def agg_kernel(h_hbm, src_hbm, dst_hbm, out_hbm, srcall_v,
                 dsta_v, dstb_v, dstc_v, rows_a, rows_b, rows_c, acc_sh,
                 sem_a, sem_b, sem_c, sem_da, sem_db, sem_dc):
    cid = lax.axis_index("c")
    sid = lax.axis_index("s")
    wid = sid * NC + cid

    ebase = wid * EPWP
    init_cp = pltpu.async_copy(h_hbm.at[pl.ds(sid * RPT, RPT)],
                               acc_sh.at[pl.ds(sid * RPT, RPT)], sem_a)
    pre_cp = pltpu.async_copy(src_hbm.at[pl.ds(ebase, EPWP)], srcall_v, sem_b)

    @pl.when(sid == 0)
    def _():
      pltpu.async_copy(h_hbm.at[pl.ds(REM_OFF, REM)],
                       acc_sh.at[pl.ds(REM_OFF, REM)], sem_c).wait()

    init_cp.wait()
    pre_cp.wait()
    plsc.subcore_barrier()

    def gather(c, rows, sem):
      pltpu.async_copy(h_hbm.at[srcall_v.at[pl.ds(c * CH, CH)]], rows, sem)

    def dfetch(c, dstv, sem):
      pltpu.async_copy(dst_hbm.at[pl.ds(ebase + c * CH, CH)], dstv, sem)

    def scat(rows, dstv, sem, dsem):
      pltpu.make_async_copy(dst_hbm.at[pl.ds(0, CH)], dstv, dsem).wait()
      pltpu.make_async_copy(h_hbm.at[pl.ds(0, CH)], rows, sem).wait()
      pltpu.sync_copy(rows, acc_sh.at[dstv], add=True)

    dfetch(0, dsta_v, sem_da)
    gather(0, rows_a, sem_a)
    dfetch(1, dstb_v, sem_db)
    gather(1, rows_b, sem_b)

    def body(g, carry):
      c = 3 * g
      dfetch(c + 2, dstc_v, sem_dc)
      gather(c + 2, rows_c, sem_c)
      scat(rows_a, dsta_v, sem_a, sem_da)
      dfetch(c + 3, dsta_v, sem_da)
      gather(c + 3, rows_a, sem_a)
      scat(rows_b, dstb_v, sem_b, sem_db)
      dfetch(c + 4, dstb_v, sem_db)
      gather(c + 4, rows_b, sem_b)
      scat(rows_c, dstc_v, sem_c, sem_dc)
      return carry

    lax.fori_loop(0, NCH // 3, body, 0)
    scat(rows_a, dsta_v, sem_a, sem_da)
    scat(rows_b, dstb_v, sem_b, sem_db)
    plsc.subcore_barrier()

    pltpu.sync_copy(acc_sh.at[pl.ds(sid * RPT, RPT)],
                    out_hbm.at[cid, pl.ds(sid * RPT, RPT)])

    @pl.when(sid == 0)
    def _():
      pltpu.sync_copy(acc_sh.at[pl.ds(REM_OFF, REM)],
                      out_hbm.at[cid, pl.ds(REM_OFF, REM)])

  return agg_kernel(h, srcp, dstp)


def _tc_layer(x, p, W, b, final):
  BR = 2000

  def body(x_ref, p_ref, w_ref, bias_ref, o_ref):
    rst = p_ref[0] + p_ref[1] - x_ref[...]
    y = jnp.dot(rst, w_ref[...], preferred_element_type=jnp.float32)
    y = jnp.maximum(y + bias_ref[...], 0.0)
    if final:
      m = jnp.max(y, axis=-1, keepdims=True)
      s = jnp.sum(jnp.exp(y - m), axis=-1, keepdims=True)
      y = y - (m + jnp.log(s))
    o_ref[...] = y

  row_spec = pl.BlockSpec((BR, D), lambda i: (i, 0))
  return pl.pallas_call(
      body,
      grid=(N // BR,),
      in_specs=[
          row_spec,
          pl.BlockSpec((NC, BR, D), lambda i: (0, i, 0)),
          pl.BlockSpec((D, D), lambda i: (0, 0)),
          pl.BlockSpec((1, D), lambda i: (0, 0)),
      ],
      out_specs=row_spec,
      out_shape=jax.ShapeDtypeStruct((N, D), jnp.float32),
  )(x, p, W, b)


def kernel(h, edge_index, W1, b1, W2, b2):
  srcp = edge_index[0]
  dstp = edge_index[1]
  b1r = b1.reshape(1, D)
  b2r = b2.reshape(1, D)

  p = _sc_aggregate(h, srcp, dstp)
  h1 = _tc_layer(h, p, W1, b1r, final=False)
  p2 = _sc_aggregate(h1, srcp, dstp)
  return _tc_layer(h1, p2, W2, b2r, final=True)
